# baseline (speedup 1.0000x reference)
import functools

import jax
import jax.numpy as jnp
from jax import lax
from jax.experimental import pallas as pl
from jax.experimental.pallas import tpu as pltpu
from jax.experimental.pallas import tpu_sc as plsc

K_NB = 32
_NEG = -3.0e38
_POS = 3.0e38


def _knn_body(xt_ref, x_ref, idx_ref, score_ref):
    xt = xt_ref[0]
    xb = x_ref[0]
    sqm = jnp.sum(xb * xb, axis=0, keepdims=True)
    inner = jnp.dot(xt, xb, preferred_element_type=jnp.float32)
    score_ref[...] = 2.0 * inner - sqm
    TN, N = score_ref.shape
    mprev = jnp.full((TN, 1), _POS, jnp.float32)
    for j in range(K_NB):
        masked = jnp.where(score_ref[...] >= mprev, _NEG, score_ref[...])
        am = jnp.argmax(masked, axis=1, keepdims=True).astype(jnp.int32)
        mprev = jnp.max(masked, axis=1, keepdims=True)
        idx_ref[0, :, pl.dslice(j, 1)] = am


def _knn(x, xt):
    B, C, N = x.shape
    TN = 512
    return pl.pallas_call(
        _knn_body,
        grid=(B, N // TN),
        in_specs=[
            pl.BlockSpec((1, TN, C), lambda b, t: (b, t, 0)),
            pl.BlockSpec((1, C, N), lambda b, t: (b, 0, 0)),
        ],
        out_specs=pl.BlockSpec((1, TN, K_NB), lambda b, t: (b, t, 0)),
        out_shape=jax.ShapeDtypeStruct((B, N, K_NB), jnp.int32),
        scratch_shapes=[pltpu.VMEM((TN, N), jnp.float32)],
    )(xt, x)


def _sc_gather(x, idx):
    B, C, N = x.shape
    mesh = plsc.VectorSubcoreMesh(core_axis_name="c", subcore_axis_name="s")
    kpw = K_NB // 4
    xf = jnp.reshape(x, (B, C * N))
    idxf = jnp.reshape(idx, (B, N * K_NB))

    @functools.partial(
        pl.kernel,
        mesh=mesh,
        compiler_params=pltpu.CompilerParams(needs_layout_passes=False),
        out_type=jax.ShapeDtypeStruct((B, K_NB, C * N), jnp.float32),
        scratch_types=[
            pltpu.VMEM((C * N,), jnp.float32),
            pltpu.VMEM((N * K_NB,), jnp.int32),
            pltpu.VMEM((C * N,), jnp.float32),
        ],
    )
    def run(x_hbm, idx_hbm, out_hbm, xv, idxv, ob):
        wid = lax.axis_index("s") * 2 + lax.axis_index("c")
        b = wid // 4
        w = wid % 4
        pltpu.sync_copy(x_hbm.at[b], xv)
        pltpu.sync_copy(idx_hbm.at[b], idxv)
        for kl in range(kpw):
            kk = w * kpw + kl

            def chunk(i, _, kk=kk):
                nvec = lax.iota(jnp.int32, 16) + i * 16
                g = plsc.load_gather(idxv, [nvec * K_NB + kk])
                for c in range(C):
                    xm = plsc.load_gather(xv, [g + c * N])
                    xn = xv[pl.dslice(c * N + i * 16, 16)]
                    ob[pl.dslice(c * N + i * 16, 16)] = xm - xn
                return 0

            lax.fori_loop(0, N // 16, chunk, 0)
            pltpu.sync_copy(ob, out_hbm.at[b, kk])

    return jnp.reshape(run(xf, idxf), (B, K_NB, C, N))


def _stats1_body(x_ref, diff_ref, m_ref):
    xb = x_ref[0]
    N = xb.shape[1]
    ones = jnp.ones((1, N), jnp.float32)
    zeros = jnp.zeros((1, N), jnp.float32)

    def body(k, M):
        hp = jnp.concatenate([xb, diff_ref[0, k], ones, zeros], axis=0)
        return M + lax.dot_general(hp, hp, (((1,), (1,)), ((), ())),
                                   preferred_element_type=jnp.float32)

    m_ref[0] = lax.fori_loop(0, K_NB, body, jnp.zeros((8, 8), jnp.float32))


def _stats1(x, diff):
    B, C, N = x.shape
    return pl.pallas_call(
        _stats1_body,
        grid=(B,),
        in_specs=[
            pl.BlockSpec((1, C, N), lambda b: (b, 0, 0)),
            pl.BlockSpec((1, K_NB, C, N), lambda b: (b, 0, 0, 0)),
        ],
        out_specs=pl.BlockSpec((1, 8, 8), lambda b: (b, 0, 0)),
        out_shape=jax.ShapeDtypeStruct((B, 8, 8), jnp.float32),
    )(x, diff)


def _main_body(cnt, x_ref, diff_ref, m1_ref, w1_ref, w2_ref, pk_ref,
               ymax_ref, ymin_ref, p2_ref):
    M = jnp.sum(m1_ref[...], axis=0)
    Shh = M[0:6, 0:6]
    sh = M[0:6, 6:7]
    W1 = w1_ref[...]
    mean1 = jnp.dot(W1, sh / cnt, preferred_element_type=jnp.float32)
    A = jnp.dot(W1, Shh / cnt, preferred_element_type=jnp.float32)
    var1 = jnp.sum(A * W1, axis=1, keepdims=True) - mean1 * mean1
    g1 = pk_ref[:, 0:1]
    b1 = pk_ref[:, 1:2]
    scale1 = g1 * lax.rsqrt(var1 + 1e-5)
    sh1 = b1 - mean1 * scale1
    W1s = W1 * scale1
    xb = x_ref[0]
    y1x = jnp.dot(W1s[:, 0:3], xb, preferred_element_type=jnp.float32) + sh1
    W2 = w2_ref[...]
    W1b = W1s[:, 3:6]
    ymax_ref[0] = jnp.full_like(ymax_ref[0], _NEG)
    ymin_ref[0] = jnp.full_like(ymin_ref[0], _POS)

    def body(k, c):
        s2, ss2 = c
        y1 = y1x + jnp.dot(W1b, diff_ref[0, k], preferred_element_type=jnp.float32)
        a1 = jnp.where(y1 > 0, y1, 0.2 * y1)
        y2 = jnp.dot(W2, a1, preferred_element_type=jnp.float32)
        ymax_ref[0] = jnp.maximum(ymax_ref[0], y2)
        ymin_ref[0] = jnp.minimum(ymin_ref[0], y2)
        return (s2 + jnp.sum(y2, axis=1, keepdims=True),
                ss2 + jnp.sum(y2 * y2, axis=1, keepdims=True))

    z = jnp.zeros((64, 1), jnp.float32)
    s2, ss2 = lax.fori_loop(0, K_NB, body, (z, z))
    p2_ref[0] = jnp.zeros_like(p2_ref[0])
    p2_ref[0, :, 0:1] = s2
    p2_ref[0, :, 1:2] = ss2


def _main(x, diff, m1, W1, W2, pk):
    B, C, N = x.shape
    cnt = float(B * N * K_NB)
    return pl.pallas_call(
        functools.partial(_main_body, cnt),
        grid=(B,),
        in_specs=[
            pl.BlockSpec((1, C, N), lambda b: (b, 0, 0)),
            pl.BlockSpec((1, K_NB, C, N), lambda b: (b, 0, 0, 0)),
            pl.BlockSpec((B, 8, 8), lambda b: (0, 0, 0)),
            pl.BlockSpec((64, 6), lambda b: (0, 0)),
            pl.BlockSpec((64, 64), lambda b: (0, 0)),
            pl.BlockSpec((64, 4), lambda b: (0, 0)),
        ],
        out_specs=[
            pl.BlockSpec((1, 64, N), lambda b: (b, 0, 0)),
            pl.BlockSpec((1, 64, N), lambda b: (b, 0, 0)),
            pl.BlockSpec((1, 64, 8), lambda b: (b, 0, 0)),
        ],
        out_shape=[
            jax.ShapeDtypeStruct((B, 64, N), jnp.float32),
            jax.ShapeDtypeStruct((B, 64, N), jnp.float32),
            jax.ShapeDtypeStruct((B, 64, 8), jnp.float32),
        ],
    )(x, diff, m1, W1, W2, pk)


def _fin_body(cnt, p2_ref, pk_ref, ymax_ref, ymin_ref, o_ref):
    s2 = jnp.sum(p2_ref[:, :, 0:1], axis=0)
    ss2 = jnp.sum(p2_ref[:, :, 1:2], axis=0)
    mean2 = s2 / cnt
    var2 = ss2 / cnt - mean2 * mean2
    g2 = pk_ref[:, 2:3]
    b2 = pk_ref[:, 3:4]
    scale2 = g2 * lax.rsqrt(var2 + 1e-5)
    sh2 = b2 - mean2 * scale2
    ysel = jnp.where(scale2 >= 0, ymax_ref[0], ymin_ref[0])
    y = ysel * scale2 + sh2
    o_ref[0] = jnp.where(y > 0, y, 0.2 * y)


def _fin(p2, pk, ymax, ymin):
    B, _, N = ymax.shape
    cnt = float(B * N * K_NB)
    return pl.pallas_call(
        functools.partial(_fin_body, cnt),
        grid=(B,),
        in_specs=[
            pl.BlockSpec((B, 64, 8), lambda b: (0, 0, 0)),
            pl.BlockSpec((64, 4), lambda b: (0, 0)),
            pl.BlockSpec((1, 64, N), lambda b: (b, 0, 0)),
            pl.BlockSpec((1, 64, N), lambda b: (b, 0, 0)),
        ],
        out_specs=pl.BlockSpec((1, 64, N), lambda b: (b, 0, 0)),
        out_shape=jax.ShapeDtypeStruct((B, 64, N), jnp.float32),
    )(p2, pk, ymax, ymin)


def kernel(x, W1, g1, b1, W2, g2, b2):
    xt = jnp.transpose(x, (0, 2, 1))
    idx = _knn(x, xt)
    diff = _sc_gather(x, idx)
    m1 = _stats1(x, diff)
    pk = jnp.stack([g1, b1, g2, b2], axis=1)
    ymax, ymin, p2 = _main(x, diff, m1, W1, W2, pk)
    return _fin(p2, pk, ymax, ymin)


# speedup vs baseline: 1.5460x; 1.5460x over previous
import functools

import jax
import jax.numpy as jnp
from jax import lax
from jax.experimental import pallas as pl
from jax.experimental.pallas import tpu as pltpu
from jax.experimental.pallas import tpu_sc as plsc

K_NB = 32
_NEG = -3.0e38
_POS = 3.0e38


def _knn_body(xt_ref, x_ref, idx_ref, score_ref):
    xt = xt_ref[0]
    xb = x_ref[0]
    sqm = jnp.sum(xb * xb, axis=0, keepdims=True)
    inner = jnp.dot(xt, xb, preferred_element_type=jnp.float32)
    score_ref[...] = 2.0 * inner - sqm
    TN, N = score_ref.shape
    lane = lax.broadcasted_iota(jnp.int32, (TN, N), 1)
    for j in range(K_NB):
        s = score_ref[...]
        am = jnp.argmax(s, axis=1, keepdims=True).astype(jnp.int32)
        score_ref[...] = jnp.where(lane == am, _NEG, s)
        idx_ref[0, :, pl.dslice(j, 1)] = am


def _knn(x, xt):
    B, C, N = x.shape
    TN = 1024
    return pl.pallas_call(
        _knn_body,
        grid=(B, N // TN),
        in_specs=[
            pl.BlockSpec((1, TN, C), lambda b, t: (b, t, 0)),
            pl.BlockSpec((1, C, N), lambda b, t: (b, 0, 0)),
        ],
        out_specs=pl.BlockSpec((1, TN, K_NB), lambda b, t: (b, t, 0)),
        out_shape=jax.ShapeDtypeStruct((B, N, K_NB), jnp.int32),
        scratch_shapes=[pltpu.VMEM((TN, N), jnp.float32)],
    )(xt, x)


def _sc_gather(x, idx):
    B, C, N = x.shape
    mesh = plsc.VectorSubcoreMesh(core_axis_name="c", subcore_axis_name="s")
    kpw = K_NB // 4
    xf = jnp.reshape(x, (B, C * N))
    idxf = jnp.reshape(idx, (B, N * K_NB))

    @functools.partial(
        pl.kernel,
        mesh=mesh,
        compiler_params=pltpu.CompilerParams(needs_layout_passes=False),
        out_type=jax.ShapeDtypeStruct((B, K_NB, C * N), jnp.float32),
        scratch_types=[
            pltpu.VMEM((C * N,), jnp.float32),
            pltpu.VMEM((N * K_NB,), jnp.int32),
            pltpu.VMEM((C * N,), jnp.float32),
        ],
    )
    def run(x_hbm, idx_hbm, out_hbm, xv, idxv, ob):
        wid = lax.axis_index("s") * 2 + lax.axis_index("c")
        b = wid // 4
        w = wid % 4
        pltpu.sync_copy(x_hbm.at[b], xv)
        pltpu.sync_copy(idx_hbm.at[b], idxv)
        for kl in range(kpw):
            kk = w * kpw + kl

            def chunk(i, _, kk=kk):
                nvec = lax.iota(jnp.int32, 16) + i * 16
                g = plsc.load_gather(idxv, [nvec * K_NB + kk])
                for c in range(C):
                    xm = plsc.load_gather(xv, [g + c * N])
                    xn = xv[pl.dslice(c * N + i * 16, 16)]
                    ob[pl.dslice(c * N + i * 16, 16)] = xm - xn
                return 0

            lax.fori_loop(0, N // 16, chunk, 0)
            pltpu.sync_copy(ob, out_hbm.at[b, kk])

    return jnp.reshape(run(xf, idxf), (B, K_NB, C, N))


def _stats1_body(x_ref, diff_ref, m_ref):
    xb = x_ref[0]
    N = xb.shape[1]
    ones = jnp.ones((1, N), jnp.float32)
    zeros = jnp.zeros((1, N), jnp.float32)

    def body(k, M):
        hp = jnp.concatenate([xb, diff_ref[0, k], ones, zeros], axis=0)
        return M + lax.dot_general(hp, hp, (((1,), (1,)), ((), ())),
                                   preferred_element_type=jnp.float32)

    m_ref[0] = lax.fori_loop(0, K_NB, body, jnp.zeros((8, 8), jnp.float32))


def _stats1(x, diff):
    B, C, N = x.shape
    return pl.pallas_call(
        _stats1_body,
        grid=(B,),
        in_specs=[
            pl.BlockSpec((1, C, N), lambda b: (b, 0, 0)),
            pl.BlockSpec((1, K_NB, C, N), lambda b: (b, 0, 0, 0)),
        ],
        out_specs=pl.BlockSpec((1, 8, 8), lambda b: (b, 0, 0)),
        out_shape=jax.ShapeDtypeStruct((B, 8, 8), jnp.float32),
    )(x, diff)


def _main_body(cnt, x_ref, diff_ref, m1_ref, w1_ref, w2_ref, pk_ref,
               ymax_ref, ymin_ref, p2_ref):
    M = jnp.sum(m1_ref[...], axis=0)
    Shh = M[0:6, 0:6]
    sh = M[0:6, 6:7]
    W1 = w1_ref[...]
    mean1 = jnp.dot(W1, sh / cnt, preferred_element_type=jnp.float32)
    A = jnp.dot(W1, Shh / cnt, preferred_element_type=jnp.float32)
    var1 = jnp.sum(A * W1, axis=1, keepdims=True) - mean1 * mean1
    g1 = pk_ref[:, 0:1]
    b1 = pk_ref[:, 1:2]
    scale1 = g1 * lax.rsqrt(var1 + 1e-5)
    sh1 = b1 - mean1 * scale1
    W1s = W1 * scale1
    xb = x_ref[0]
    y1x = jnp.dot(W1s[:, 0:3], xb, preferred_element_type=jnp.float32) + sh1
    W2 = w2_ref[...]
    W1b = W1s[:, 3:6]
    ymax_ref[0] = jnp.full_like(ymax_ref[0], _NEG)
    ymin_ref[0] = jnp.full_like(ymin_ref[0], _POS)

    def body(k, c):
        s2, ss2 = c
        y1 = y1x + jnp.dot(W1b, diff_ref[0, k], preferred_element_type=jnp.float32)
        a1 = jnp.where(y1 > 0, y1, 0.2 * y1)
        y2 = jnp.dot(W2, a1, preferred_element_type=jnp.float32)
        ymax_ref[0] = jnp.maximum(ymax_ref[0], y2)
        ymin_ref[0] = jnp.minimum(ymin_ref[0], y2)
        return (s2 + jnp.sum(y2, axis=1, keepdims=True),
                ss2 + jnp.sum(y2 * y2, axis=1, keepdims=True))

    z = jnp.zeros((64, 1), jnp.float32)
    s2, ss2 = lax.fori_loop(0, K_NB, body, (z, z))
    p2_ref[0] = jnp.zeros_like(p2_ref[0])
    p2_ref[0, :, 0:1] = s2
    p2_ref[0, :, 1:2] = ss2


def _main(x, diff, m1, W1, W2, pk):
    B, C, N = x.shape
    cnt = float(B * N * K_NB)
    return pl.pallas_call(
        functools.partial(_main_body, cnt),
        grid=(B,),
        in_specs=[
            pl.BlockSpec((1, C, N), lambda b: (b, 0, 0)),
            pl.BlockSpec((1, K_NB, C, N), lambda b: (b, 0, 0, 0)),
            pl.BlockSpec((B, 8, 8), lambda b: (0, 0, 0)),
            pl.BlockSpec((64, 6), lambda b: (0, 0)),
            pl.BlockSpec((64, 64), lambda b: (0, 0)),
            pl.BlockSpec((64, 4), lambda b: (0, 0)),
        ],
        out_specs=[
            pl.BlockSpec((1, 64, N), lambda b: (b, 0, 0)),
            pl.BlockSpec((1, 64, N), lambda b: (b, 0, 0)),
            pl.BlockSpec((1, 64, 8), lambda b: (b, 0, 0)),
        ],
        out_shape=[
            jax.ShapeDtypeStruct((B, 64, N), jnp.float32),
            jax.ShapeDtypeStruct((B, 64, N), jnp.float32),
            jax.ShapeDtypeStruct((B, 64, 8), jnp.float32),
        ],
    )(x, diff, m1, W1, W2, pk)


def _fin_body(cnt, p2_ref, pk_ref, ymax_ref, ymin_ref, o_ref):
    s2 = jnp.sum(p2_ref[:, :, 0:1], axis=0)
    ss2 = jnp.sum(p2_ref[:, :, 1:2], axis=0)
    mean2 = s2 / cnt
    var2 = ss2 / cnt - mean2 * mean2
    g2 = pk_ref[:, 2:3]
    b2 = pk_ref[:, 3:4]
    scale2 = g2 * lax.rsqrt(var2 + 1e-5)
    sh2 = b2 - mean2 * scale2
    ysel = jnp.where(scale2 >= 0, ymax_ref[0], ymin_ref[0])
    y = ysel * scale2 + sh2
    o_ref[0] = jnp.where(y > 0, y, 0.2 * y)


def _fin(p2, pk, ymax, ymin):
    B, _, N = ymax.shape
    cnt = float(B * N * K_NB)
    return pl.pallas_call(
        functools.partial(_fin_body, cnt),
        grid=(B,),
        in_specs=[
            pl.BlockSpec((B, 64, 8), lambda b: (0, 0, 0)),
            pl.BlockSpec((64, 4), lambda b: (0, 0)),
            pl.BlockSpec((1, 64, N), lambda b: (b, 0, 0)),
            pl.BlockSpec((1, 64, N), lambda b: (b, 0, 0)),
        ],
        out_specs=pl.BlockSpec((1, 64, N), lambda b: (b, 0, 0)),
        out_shape=jax.ShapeDtypeStruct((B, 64, N), jnp.float32),
    )(p2, pk, ymax, ymin)


def kernel(x, W1, g1, b1, W2, g2, b2):
    xt = jnp.transpose(x, (0, 2, 1))
    idx = _knn(x, xt)
    diff = _sc_gather(x, idx)
    m1 = _stats1(x, diff)
    pk = jnp.stack([g1, b1, g2, b2], axis=1)
    ymax, ymin, p2 = _main(x, diff, m1, W1, W2, pk)
    return _fin(p2, pk, ymax, ymin)


# speedup vs baseline: 1.7252x; 1.1159x over previous
import functools

import jax
import jax.numpy as jnp
from jax import lax
from jax.experimental import pallas as pl
from jax.experimental.pallas import tpu as pltpu
from jax.experimental.pallas import tpu_sc as plsc

K_NB = 32
_NEG = -3.0e38
_POS = 3.0e38


def _knn_body(xt_ref, x_ref, idx_ref, score_ref):
    xt = xt_ref[0]
    xb = x_ref[0]
    sqm = jnp.sum(xb * xb, axis=0, keepdims=True)
    inner = jnp.dot(xt, xb, preferred_element_type=jnp.float32)
    score_ref[...] = 2.0 * inner - sqm
    TN, N = score_ref.shape
    lane = lax.broadcasted_iota(jnp.int32, (TN, N), 1)
    for j in range(K_NB):
        s = score_ref[...]
        am = jnp.argmax(s, axis=1, keepdims=True).astype(jnp.int32)
        score_ref[...] = jnp.where(lane == am, _NEG, s)
        idx_ref[0, :, pl.dslice(j, 1)] = am


def _knn(x, xt):
    B, C, N = x.shape
    TN = 256
    return pl.pallas_call(
        _knn_body,
        grid=(B, N // TN),
        in_specs=[
            pl.BlockSpec((1, TN, C), lambda b, t: (b, t, 0)),
            pl.BlockSpec((1, C, N), lambda b, t: (b, 0, 0)),
        ],
        out_specs=pl.BlockSpec((1, TN, K_NB), lambda b, t: (b, t, 0)),
        out_shape=jax.ShapeDtypeStruct((B, N, K_NB), jnp.int32),
        scratch_shapes=[pltpu.VMEM((TN, N), jnp.float32)],
    )(xt, x)


def _sc_gather(x, idx):
    B, C, N = x.shape
    mesh = plsc.VectorSubcoreMesh(core_axis_name="c", subcore_axis_name="s")
    kpw = K_NB // 4
    xf = jnp.reshape(x, (B, C * N))
    idxf = jnp.reshape(idx, (B, N * K_NB))

    @functools.partial(
        pl.kernel,
        mesh=mesh,
        compiler_params=pltpu.CompilerParams(needs_layout_passes=False),
        out_type=jax.ShapeDtypeStruct((B, K_NB, C * N), jnp.float32),
        scratch_types=[
            pltpu.VMEM((C * N,), jnp.float32),
            pltpu.VMEM((N * K_NB,), jnp.int32),
            pltpu.VMEM((C * N,), jnp.float32),
        ],
    )
    def run(x_hbm, idx_hbm, out_hbm, xv, idxv, ob):
        wid = lax.axis_index("s") * 2 + lax.axis_index("c")
        b = wid // 4
        w = wid % 4
        pltpu.sync_copy(x_hbm.at[b], xv)
        pltpu.sync_copy(idx_hbm.at[b], idxv)
        for kl in range(kpw):
            kk = w * kpw + kl

            def chunk(i, _, kk=kk):
                nvec = lax.iota(jnp.int32, 16) + i * 16
                g = plsc.load_gather(idxv, [nvec * K_NB + kk])
                for c in range(C):
                    xm = plsc.load_gather(xv, [g + c * N])
                    xn = xv[pl.dslice(c * N + i * 16, 16)]
                    ob[pl.dslice(c * N + i * 16, 16)] = xm - xn
                return 0

            lax.fori_loop(0, N // 16, chunk, 0)
            pltpu.sync_copy(ob, out_hbm.at[b, kk])

    return jnp.reshape(run(xf, idxf), (B, K_NB, C, N))


def _stats1_body(x_ref, diff_ref, m_ref):
    xb = x_ref[0]
    N = xb.shape[1]
    ones = jnp.ones((1, N), jnp.float32)
    zeros = jnp.zeros((1, N), jnp.float32)

    def body(k, M):
        hp = jnp.concatenate([xb, diff_ref[0, k], ones, zeros], axis=0)
        return M + lax.dot_general(hp, hp, (((1,), (1,)), ((), ())),
                                   preferred_element_type=jnp.float32)

    m_ref[0] = lax.fori_loop(0, K_NB, body, jnp.zeros((8, 8), jnp.float32))


def _stats1(x, diff):
    B, C, N = x.shape
    return pl.pallas_call(
        _stats1_body,
        grid=(B,),
        in_specs=[
            pl.BlockSpec((1, C, N), lambda b: (b, 0, 0)),
            pl.BlockSpec((1, K_NB, C, N), lambda b: (b, 0, 0, 0)),
        ],
        out_specs=pl.BlockSpec((1, 8, 8), lambda b: (b, 0, 0)),
        out_shape=jax.ShapeDtypeStruct((B, 8, 8), jnp.float32),
    )(x, diff)


def _main_body(cnt, x_ref, diff_ref, m1_ref, w1_ref, w2_ref, pk_ref,
               ymax_ref, ymin_ref, p2_ref):
    M = jnp.sum(m1_ref[...], axis=0)
    Shh = M[0:6, 0:6]
    sh = M[0:6, 6:7]
    W1 = w1_ref[...]
    mean1 = jnp.dot(W1, sh / cnt, preferred_element_type=jnp.float32)
    A = jnp.dot(W1, Shh / cnt, preferred_element_type=jnp.float32)
    var1 = jnp.sum(A * W1, axis=1, keepdims=True) - mean1 * mean1
    g1 = pk_ref[:, 0:1]
    b1 = pk_ref[:, 1:2]
    scale1 = g1 * lax.rsqrt(var1 + 1e-5)
    sh1 = b1 - mean1 * scale1
    W1s = W1 * scale1
    xb = x_ref[0]
    y1x = jnp.dot(W1s[:, 0:3], xb, preferred_element_type=jnp.float32) + sh1
    W2 = w2_ref[...]
    W1b = W1s[:, 3:6]
    ymax_ref[0] = jnp.full_like(ymax_ref[0], _NEG)
    ymin_ref[0] = jnp.full_like(ymin_ref[0], _POS)

    def body(k, c):
        s2, ss2 = c
        y1 = y1x + jnp.dot(W1b, diff_ref[0, k], preferred_element_type=jnp.float32)
        a1 = jnp.where(y1 > 0, y1, 0.2 * y1)
        y2 = jnp.dot(W2, a1, preferred_element_type=jnp.float32)
        ymax_ref[0] = jnp.maximum(ymax_ref[0], y2)
        ymin_ref[0] = jnp.minimum(ymin_ref[0], y2)
        return (s2 + jnp.sum(y2, axis=1, keepdims=True),
                ss2 + jnp.sum(y2 * y2, axis=1, keepdims=True))

    z = jnp.zeros((64, 1), jnp.float32)
    s2, ss2 = lax.fori_loop(0, K_NB, body, (z, z))
    p2_ref[0] = jnp.zeros_like(p2_ref[0])
    p2_ref[0, :, 0:1] = s2
    p2_ref[0, :, 1:2] = ss2


def _main(x, diff, m1, W1, W2, pk):
    B, C, N = x.shape
    cnt = float(B * N * K_NB)
    return pl.pallas_call(
        functools.partial(_main_body, cnt),
        grid=(B,),
        in_specs=[
            pl.BlockSpec((1, C, N), lambda b: (b, 0, 0)),
            pl.BlockSpec((1, K_NB, C, N), lambda b: (b, 0, 0, 0)),
            pl.BlockSpec((B, 8, 8), lambda b: (0, 0, 0)),
            pl.BlockSpec((64, 6), lambda b: (0, 0)),
            pl.BlockSpec((64, 64), lambda b: (0, 0)),
            pl.BlockSpec((64, 4), lambda b: (0, 0)),
        ],
        out_specs=[
            pl.BlockSpec((1, 64, N), lambda b: (b, 0, 0)),
            pl.BlockSpec((1, 64, N), lambda b: (b, 0, 0)),
            pl.BlockSpec((1, 64, 8), lambda b: (b, 0, 0)),
        ],
        out_shape=[
            jax.ShapeDtypeStruct((B, 64, N), jnp.float32),
            jax.ShapeDtypeStruct((B, 64, N), jnp.float32),
            jax.ShapeDtypeStruct((B, 64, 8), jnp.float32),
        ],
    )(x, diff, m1, W1, W2, pk)


def _fin_body(cnt, p2_ref, pk_ref, ymax_ref, ymin_ref, o_ref):
    s2 = jnp.sum(p2_ref[:, :, 0:1], axis=0)
    ss2 = jnp.sum(p2_ref[:, :, 1:2], axis=0)
    mean2 = s2 / cnt
    var2 = ss2 / cnt - mean2 * mean2
    g2 = pk_ref[:, 2:3]
    b2 = pk_ref[:, 3:4]
    scale2 = g2 * lax.rsqrt(var2 + 1e-5)
    sh2 = b2 - mean2 * scale2
    ysel = jnp.where(scale2 >= 0, ymax_ref[0], ymin_ref[0])
    y = ysel * scale2 + sh2
    o_ref[0] = jnp.where(y > 0, y, 0.2 * y)


def _fin(p2, pk, ymax, ymin):
    B, _, N = ymax.shape
    cnt = float(B * N * K_NB)
    return pl.pallas_call(
        functools.partial(_fin_body, cnt),
        grid=(B,),
        in_specs=[
            pl.BlockSpec((B, 64, 8), lambda b: (0, 0, 0)),
            pl.BlockSpec((64, 4), lambda b: (0, 0)),
            pl.BlockSpec((1, 64, N), lambda b: (b, 0, 0)),
            pl.BlockSpec((1, 64, N), lambda b: (b, 0, 0)),
        ],
        out_specs=pl.BlockSpec((1, 64, N), lambda b: (b, 0, 0)),
        out_shape=jax.ShapeDtypeStruct((B, 64, N), jnp.float32),
    )(p2, pk, ymax, ymin)


def kernel(x, W1, g1, b1, W2, g2, b2):
    xt = jnp.transpose(x, (0, 2, 1))
    idx = _knn(x, xt)
    diff = _sc_gather(x, idx)
    m1 = _stats1(x, diff)
    pk = jnp.stack([g1, b1, g2, b2], axis=1)
    ymax, ymin, p2 = _main(x, diff, m1, W1, W2, pk)
    return _fin(p2, pk, ymax, ymin)


# speedup vs baseline: 1.7952x; 1.0406x over previous
import functools

import jax
import jax.numpy as jnp
from jax import lax
from jax.experimental import pallas as pl
from jax.experimental.pallas import tpu as pltpu
from jax.experimental.pallas import tpu_sc as plsc

K_NB = 32
_NEG = -3.0e38
_POS = 3.0e38


def _knn_body(xt_ref, x_ref, idx_ref, score_ref):
    xt = xt_ref[0]
    xb = x_ref[0]
    sqm = jnp.sum(xb * xb, axis=0, keepdims=True)
    inner = jnp.dot(xt, xb, preferred_element_type=jnp.float32)
    score_ref[...] = 2.0 * inner - sqm
    TN, N = score_ref.shape
    lane = lax.broadcasted_iota(jnp.int32, (TN, N), 1)
    for j in range(K_NB):
        s = score_ref[...]
        am = jnp.argmax(s, axis=1, keepdims=True).astype(jnp.int32)
        score_ref[...] = jnp.where(lane == am, _NEG, s)
        idx_ref[0, :, pl.dslice(j, 1)] = am


def _knn(x, xt):
    B, C, N = x.shape
    TN = 512
    return pl.pallas_call(
        _knn_body,
        grid=(B, N // TN),
        in_specs=[
            pl.BlockSpec((1, TN, C), lambda b, t: (b, t, 0)),
            pl.BlockSpec((1, C, N), lambda b, t: (b, 0, 0)),
        ],
        out_specs=pl.BlockSpec((1, TN, K_NB), lambda b, t: (b, t, 0)),
        out_shape=jax.ShapeDtypeStruct((B, N, K_NB), jnp.int32),
        scratch_shapes=[pltpu.VMEM((TN, N), jnp.float32)],
    )(xt, x)


def _sc_gather(x, idx):
    B, C, N = x.shape
    mesh = plsc.VectorSubcoreMesh(core_axis_name="c", subcore_axis_name="s")
    kpw = K_NB // 4
    xf = jnp.reshape(x, (B, C * N))
    idxf = jnp.reshape(idx, (B, N * K_NB))

    @functools.partial(
        pl.kernel,
        mesh=mesh,
        compiler_params=pltpu.CompilerParams(needs_layout_passes=False),
        out_type=jax.ShapeDtypeStruct((B, K_NB, C * N), jnp.float32),
        scratch_types=[
            pltpu.VMEM((C * N,), jnp.float32),
            pltpu.VMEM((N * K_NB,), jnp.int32),
            pltpu.VMEM((C * N,), jnp.float32),
        ],
    )
    def run(x_hbm, idx_hbm, out_hbm, xv, idxv, ob):
        wid = lax.axis_index("s") * 2 + lax.axis_index("c")
        b = wid // 4
        w = wid % 4
        pltpu.sync_copy(x_hbm.at[b], xv)
        pltpu.sync_copy(idx_hbm.at[b], idxv)
        for kl in range(kpw):
            kk = w * kpw + kl

            def chunk(i, _, kk=kk):
                nvec = lax.iota(jnp.int32, 16) + i * 16
                g = plsc.load_gather(idxv, [nvec * K_NB + kk])
                for c in range(C):
                    xm = plsc.load_gather(xv, [g + c * N])
                    xn = xv[pl.dslice(c * N + i * 16, 16)]
                    ob[pl.dslice(c * N + i * 16, 16)] = xm - xn
                return 0

            lax.fori_loop(0, N // 16, chunk, 0)
            pltpu.sync_copy(ob, out_hbm.at[b, kk])

    return jnp.reshape(run(xf, idxf), (B, K_NB, C, N))


def _stats1_body(x_ref, diff_ref, m_ref):
    xb = x_ref[0]
    N = xb.shape[1]
    ones = jnp.ones((1, N), jnp.float32)
    zeros = jnp.zeros((1, N), jnp.float32)

    def body(k, M):
        hp = jnp.concatenate([xb, diff_ref[0, k], ones, zeros], axis=0)
        return M + lax.dot_general(hp, hp, (((1,), (1,)), ((), ())),
                                   preferred_element_type=jnp.float32)

    m_ref[0] = lax.fori_loop(0, K_NB, body, jnp.zeros((8, 8), jnp.float32))


def _stats1(x, diff):
    B, C, N = x.shape
    return pl.pallas_call(
        _stats1_body,
        grid=(B,),
        in_specs=[
            pl.BlockSpec((1, C, N), lambda b: (b, 0, 0)),
            pl.BlockSpec((1, K_NB, C, N), lambda b: (b, 0, 0, 0)),
        ],
        out_specs=pl.BlockSpec((1, 8, 8), lambda b: (b, 0, 0)),
        out_shape=jax.ShapeDtypeStruct((B, 8, 8), jnp.float32),
    )(x, diff)


def _main_body(cnt, x_ref, diff_ref, m1_ref, w1_ref, w2_ref, pk_ref,
               ymax_ref, ymin_ref, p2_ref):
    M = jnp.sum(m1_ref[...], axis=0)
    Shh = M[0:6, 0:6]
    sh = M[0:6, 6:7]
    W1 = w1_ref[...]
    mean1 = jnp.dot(W1, sh / cnt, preferred_element_type=jnp.float32)
    A = jnp.dot(W1, Shh / cnt, preferred_element_type=jnp.float32)
    var1 = jnp.sum(A * W1, axis=1, keepdims=True) - mean1 * mean1
    g1 = pk_ref[:, 0:1]
    b1 = pk_ref[:, 1:2]
    scale1 = g1 * lax.rsqrt(var1 + 1e-5)
    sh1 = b1 - mean1 * scale1
    W1s = W1 * scale1
    xb = x_ref[0]
    y1x = jnp.dot(W1s[:, 0:3], xb, preferred_element_type=jnp.float32) + sh1
    W2 = w2_ref[...]
    W1b = W1s[:, 3:6]
    ymax_ref[0] = jnp.full_like(ymax_ref[0], _NEG)
    ymin_ref[0] = jnp.full_like(ymin_ref[0], _POS)

    def body(k, c):
        s2, ss2 = c
        y1 = y1x + jnp.dot(W1b, diff_ref[0, k], preferred_element_type=jnp.float32)
        a1 = jnp.where(y1 > 0, y1, 0.2 * y1)
        y2 = jnp.dot(W2, a1, preferred_element_type=jnp.float32)
        ymax_ref[0] = jnp.maximum(ymax_ref[0], y2)
        ymin_ref[0] = jnp.minimum(ymin_ref[0], y2)
        return (s2 + jnp.sum(y2, axis=1, keepdims=True),
                ss2 + jnp.sum(y2 * y2, axis=1, keepdims=True))

    z = jnp.zeros((64, 1), jnp.float32)
    s2, ss2 = lax.fori_loop(0, K_NB, body, (z, z))
    p2_ref[0] = jnp.zeros_like(p2_ref[0])
    p2_ref[0, :, 0:1] = s2
    p2_ref[0, :, 1:2] = ss2


def _main(x, diff, m1, W1, W2, pk):
    B, C, N = x.shape
    cnt = float(B * N * K_NB)
    return pl.pallas_call(
        functools.partial(_main_body, cnt),
        grid=(B,),
        in_specs=[
            pl.BlockSpec((1, C, N), lambda b: (b, 0, 0)),
            pl.BlockSpec((1, K_NB, C, N), lambda b: (b, 0, 0, 0)),
            pl.BlockSpec((B, 8, 8), lambda b: (0, 0, 0)),
            pl.BlockSpec((64, 6), lambda b: (0, 0)),
            pl.BlockSpec((64, 64), lambda b: (0, 0)),
            pl.BlockSpec((64, 4), lambda b: (0, 0)),
        ],
        out_specs=[
            pl.BlockSpec((1, 64, N), lambda b: (b, 0, 0)),
            pl.BlockSpec((1, 64, N), lambda b: (b, 0, 0)),
            pl.BlockSpec((1, 64, 8), lambda b: (b, 0, 0)),
        ],
        out_shape=[
            jax.ShapeDtypeStruct((B, 64, N), jnp.float32),
            jax.ShapeDtypeStruct((B, 64, N), jnp.float32),
            jax.ShapeDtypeStruct((B, 64, 8), jnp.float32),
        ],
    )(x, diff, m1, W1, W2, pk)


def _fin_body(cnt, p2_ref, pk_ref, ymax_ref, ymin_ref, o_ref):
    s2 = jnp.sum(p2_ref[:, :, 0:1], axis=0)
    ss2 = jnp.sum(p2_ref[:, :, 1:2], axis=0)
    mean2 = s2 / cnt
    var2 = ss2 / cnt - mean2 * mean2
    g2 = pk_ref[:, 2:3]
    b2 = pk_ref[:, 3:4]
    scale2 = g2 * lax.rsqrt(var2 + 1e-5)
    sh2 = b2 - mean2 * scale2
    ysel = jnp.where(scale2 >= 0, ymax_ref[0], ymin_ref[0])
    y = ysel * scale2 + sh2
    o_ref[0] = jnp.where(y > 0, y, 0.2 * y)


def _fin(p2, pk, ymax, ymin):
    B, _, N = ymax.shape
    cnt = float(B * N * K_NB)
    return pl.pallas_call(
        functools.partial(_fin_body, cnt),
        grid=(B,),
        in_specs=[
            pl.BlockSpec((B, 64, 8), lambda b: (0, 0, 0)),
            pl.BlockSpec((64, 4), lambda b: (0, 0)),
            pl.BlockSpec((1, 64, N), lambda b: (b, 0, 0)),
            pl.BlockSpec((1, 64, N), lambda b: (b, 0, 0)),
        ],
        out_specs=pl.BlockSpec((1, 64, N), lambda b: (b, 0, 0)),
        out_shape=jax.ShapeDtypeStruct((B, 64, N), jnp.float32),
    )(p2, pk, ymax, ymin)


def kernel(x, W1, g1, b1, W2, g2, b2):
    xt = jnp.transpose(x, (0, 2, 1))
    idx = _knn(x, xt)
    diff = _sc_gather(x, idx)
    m1 = _stats1(x, diff)
    pk = jnp.stack([g1, b1, g2, b2], axis=1)
    ymax, ymin, p2 = _main(x, diff, m1, W1, W2, pk)
    return _fin(p2, pk, ymax, ymin)
